# baseline (speedup 1.0000x reference)
import functools

import jax
import jax.numpy as jnp
from jax import lax
from jax.experimental import pallas as pl
from jax.experimental.pallas import tpu as pltpu
from jax.experimental.pallas import tpu_sc as plsc

BATCH = 4
SEQ_LEN = 2048
D_MODEL = 1024

NUM_CORES = 2
NUM_SUBCORES = 16
NUM_WORKERS = NUM_CORES * NUM_SUBCORES
SEQ_PER_W = SEQ_LEN // NUM_WORKERS
ROWS_PER_CHUNK = 16
CHUNKS = SEQ_PER_W // ROWS_PER_CHUNK
NBUF = 4
VECS = ROWS_PER_CHUNK * D_MODEL // 16
LANES_PER_ROW = D_MODEL // 16
JOBS = CHUNKS * BATCH

_mesh = plsc.VectorSubcoreMesh(core_axis_name="c", subcore_axis_name="s")

_scratch = (
    [pltpu.VMEM((SEQ_PER_W, D_MODEL), jnp.float32)]
    + [pltpu.VMEM((ROWS_PER_CHUNK, D_MODEL), jnp.float32) for _ in range(NBUF)]
    + [pltpu.SemaphoreType.DMA for _ in range(1 + 2 * NBUF)]
)


@functools.partial(
    pl.kernel,
    mesh=_mesh,
    out_type=jax.ShapeDtypeStruct((BATCH, SEQ_LEN, D_MODEL), jnp.float32),
    scratch_types=_scratch,
)
def _pos_enc_sc(x_hbm, emb_hbm, out_hbm, *refs):
    ebuf = refs[0]
    xbufs = refs[1 : 1 + NBUF]
    esem = refs[1 + NBUF]
    isems = refs[2 + NBUF : 2 + 2 * NBUF]
    osems = refs[2 + 2 * NBUF : 2 + 3 * NBUF]

    wid = lax.axis_index("s") * NUM_CORES + lax.axis_index("c")
    seq_base = wid * SEQ_PER_W

    def issue_in(j):
        c, b = divmod(j, BATCH)
        return pltpu.async_copy(
            x_hbm.at[b, pl.ds(seq_base + c * ROWS_PER_CHUNK, ROWS_PER_CHUNK)],
            xbufs[j % NBUF],
            isems[j % NBUF],
        )

    def issue_out(j):
        c, b = divmod(j, BATCH)
        return pltpu.async_copy(
            xbufs[j % NBUF],
            out_hbm.at[b, pl.ds(seq_base + c * ROWS_PER_CHUNK, ROWS_PER_CHUNK)],
            osems[j % NBUF],
        )

    ecpy = pltpu.async_copy(emb_hbm.at[pl.ds(seq_base, SEQ_PER_W)], ebuf, esem)
    incpy = [None] * JOBS
    outcpy = [None] * JOBS
    for j in range(NBUF - 1):
        incpy[j] = issue_in(j)
    ecpy.wait()

    for j in range(JOBS):
        c, b = divmod(j, BATCH)
        nj = j + NBUF - 1
        if nj < JOBS:
            if nj - NBUF >= 0:
                outcpy[nj - NBUF].wait()
            incpy[nj] = issue_in(nj)
        incpy[j].wait()
        erow = c * ROWS_PER_CHUNK

        @plsc.parallel_loop(0, VECS, step=1, unroll=8)
        def _add(i, buf=xbufs[j % NBUF], erow=erow):
            r = i // LANES_PER_ROW
            col = (i % LANES_PER_ROW) * 16
            sl = pl.ds(col, 16)
            plsc.addupdate(buf.at[r, sl], ebuf[erow + r, sl])

        outcpy[j] = issue_out(j)

    for j in range(JOBS - NBUF, JOBS):
        outcpy[j].wait()


def kernel(x, emb):
    return _pos_enc_sc(x, emb)
